# baseline (speedup 1.0000x reference)
import functools

import jax
import jax.numpy as jnp
from jax import lax
from jax.experimental import pallas as pl
from jax.experimental.pallas import tpu as pltpu
from jax.experimental.pallas import tpu_sc as plsc

_FALSE_NEG_SCORE = -1e9
_NW = 32
_W = 128


def _sc_gather_rows(table3, ids):
    n, = ids.shape
    d = table3.shape[2]
    b_per_w = n // _NW
    mesh = plsc.VectorSubcoreMesh(core_axis_name="c", subcore_axis_name="s")

    @functools.partial(
        pl.kernel,
        mesh=mesh,
        out_type=jax.ShapeDtypeStruct((n, _W), jnp.float32),
        scratch_types=[
            pltpu.VMEM((b_per_w,), jnp.int32),
            pltpu.VMEM((b_per_w, _W), jnp.float32),
            pltpu.SemaphoreType.DMA,
        ],
        compiler_params=pltpu.CompilerParams(needs_layout_passes=False),
    )
    def gather_kernel(ids_hbm, table_hbm, out_hbm, ids_v, rows_v, sem):
        wid = lax.axis_index("s") * 2 + lax.axis_index("c")
        base = wid * b_per_w
        pltpu.sync_copy(ids_hbm.at[pl.ds(base, b_per_w)], ids_v)
        lane_iota = lax.iota(jnp.int32, 16)
        id_lane = jnp.full((16,), d, jnp.int32)

        def chunk_body(c, _):
            ids16 = ids_v[pl.ds(c * 16, 16)]
            plsc.store_scatter(rows_v, [c * 16 + lane_iota, id_lane],
                               ids16.astype(jnp.float32))
            for l in range(16):
                idl = jnp.sum(jnp.where(lane_iota == l, ids16, 0))
                t = lax.shift_right_logical(idl, 3)
                s = lax.bitwise_and(idl, 7)
                pltpu.async_copy(table_hbm.at[t, s],
                                 rows_v.at[c * 16 + l, pl.ds(0, d)], sem)
            return ()

        lax.fori_loop(0, b_per_w // 16, chunk_body, (), unroll=False)
        pltpu.make_async_copy(out_hbm.at[pl.ds(base, b_per_w // 2)],
                              rows_v.at[pl.ds(0, b_per_w // 2)], sem).wait()
        pltpu.sync_copy(rows_v, out_hbm.at[pl.ds(base, b_per_w)])

    return gather_kernel(ids, table3)


def _tc_logits(emb_all, query, neg_idf, n_neg, bb):
    b, d = query.shape
    np_pad = neg_idf.shape[1]
    neg_row_block = b // np_pad

    def body(q_ref, pos_ref, neg_ref, nid_ref, out_ref):
        q = q_ref[...]
        p = pos_ref[:, :d]
        pid = pos_ref[:, d:d + 1]
        pos_score = jnp.sum(q * p, axis=1, keepdims=True)
        scores = lax.dot_general(
            q, neg_ref[:, :d], (((1,), (1,)), ((), ())),
            preferred_element_type=jnp.float32)
        mask = pid == nid_ref[...]
        scores = jnp.where(mask, _FALSE_NEG_SCORE, scores)
        out_ref[...] = jnp.concatenate(
            [pos_score, scores[:, :n_neg]], axis=1)

    return pl.pallas_call(
        body,
        grid=(b // bb,),
        in_specs=[
            pl.BlockSpec((bb, d), lambda i: (i, 0)),
            pl.BlockSpec((bb, _W), lambda i: (i, 0)),
            pl.BlockSpec((np_pad, _W), lambda i: (neg_row_block, 0)),
            pl.BlockSpec((1, np_pad), lambda i: (0, 0)),
        ],
        out_specs=pl.BlockSpec((bb, 1 + n_neg), lambda i: (i, 0)),
        out_shape=jax.ShapeDtypeStruct((b, 1 + n_neg), jnp.float32),
    )(query, emb_all, emb_all, neg_idf)


def kernel(query, table, positive_ids, negative_ids):
    b, d = query.shape
    n = negative_ids.shape[0]
    np_pad = 1024
    neg_pad = jnp.pad(negative_ids, (0, np_pad - n))
    ids_all = jnp.concatenate([positive_ids, neg_pad])
    table3 = table.reshape(table.shape[0] // 8, 8, d)
    emb_all = _sc_gather_rows(table3, ids_all)
    neg_idf = jnp.pad(negative_ids.astype(jnp.float32).reshape(1, n),
                      ((0, 0), (0, np_pad - n)), constant_values=-1.0)
    return _tc_logits(emb_all, query, neg_idf, n, bb=1024)


# speedup vs baseline: 1.0110x; 1.0110x over previous
import functools

import jax
import jax.numpy as jnp
from jax import lax
from jax.experimental import pallas as pl
from jax.experimental.pallas import tpu as pltpu
from jax.experimental.pallas import tpu_sc as plsc

_FALSE_NEG_SCORE = -1e9
_NW = 32
_W = 128


def _sc_gather_rows(table3, ids):
    n, = ids.shape
    d = table3.shape[2]
    b_per_w = n // _NW
    mesh = plsc.VectorSubcoreMesh(core_axis_name="c", subcore_axis_name="s")

    @functools.partial(
        pl.kernel,
        mesh=mesh,
        out_type=jax.ShapeDtypeStruct((n, _W), jnp.float32),
        scratch_types=[
            pltpu.VMEM((b_per_w,), jnp.int32),
            pltpu.VMEM((b_per_w, _W), jnp.float32),
            pltpu.SemaphoreType.DMA,
        ],
        compiler_params=pltpu.CompilerParams(needs_layout_passes=False),
    )
    def gather_kernel(ids_hbm, table_hbm, out_hbm, ids_v, rows_v, sem):
        wid = lax.axis_index("s") * 2 + lax.axis_index("c")
        base = wid * b_per_w
        pltpu.sync_copy(ids_hbm.at[pl.ds(base, b_per_w)], ids_v)
        lane_iota = lax.iota(jnp.int32, 16)
        id_lane = jnp.full((16,), d, jnp.int32)

        def chunk_body(c, _):
            ids16 = ids_v[pl.ds(c * 16, 16)]
            plsc.store_scatter(rows_v, [c * 16 + lane_iota, id_lane],
                               ids16.astype(jnp.float32))
            for l in range(16):
                idl = jnp.sum(jnp.where(lane_iota == l, ids16, 0))
                t = lax.shift_right_logical(idl, 3)
                s = lax.bitwise_and(idl, 7)
                pltpu.async_copy(table_hbm.at[t, s],
                                 rows_v.at[c * 16 + l, pl.ds(0, d)], sem)
            return ()

        lax.fori_loop(0, b_per_w // 16, chunk_body, (), unroll=False)
        pltpu.make_async_copy(out_hbm.at[pl.ds(base, b_per_w // 2)],
                              rows_v.at[pl.ds(0, b_per_w // 2)], sem).wait()
        pltpu.sync_copy(rows_v, out_hbm.at[pl.ds(base, b_per_w)])

    return gather_kernel(ids, table3)


def _tc_logits(emb_all, query, neg_idf, n_neg, bb):
    b, d = query.shape
    np_pad = neg_idf.shape[1]
    neg_row_block = b // np_pad

    def body(q_ref, pos_ref, neg_ref, nid_ref, out_ref):
        q = q_ref[...]
        p = pos_ref[:, :d]
        pid = pos_ref[:, d:d + 1]
        pos_score = jnp.sum(q * p, axis=1, keepdims=True)
        scores = lax.dot_general(
            q, neg_ref[:, :d], (((1,), (1,)), ((), ())),
            preferred_element_type=jnp.float32)
        mask = pid == nid_ref[...]
        scores = jnp.where(mask, _FALSE_NEG_SCORE, scores)
        out_ref[...] = jnp.concatenate(
            [pos_score, scores[:, :n_neg]], axis=1)

    return pl.pallas_call(
        body,
        grid=(b // bb,),
        in_specs=[
            pl.BlockSpec((bb, d), lambda i: (i, 0)),
            pl.BlockSpec((bb, _W), lambda i: (i, 0)),
            pl.BlockSpec((np_pad, _W), lambda i: (neg_row_block, 0)),
            pl.BlockSpec((1, np_pad), lambda i: (0, 0)),
        ],
        out_specs=pl.BlockSpec((bb, 1 + n_neg), lambda i: (i, 0)),
        out_shape=jax.ShapeDtypeStruct((b, 1 + n_neg), jnp.float32),
    )(query, emb_all, emb_all, neg_idf)


def kernel(query, table, positive_ids, negative_ids):
    b, d = query.shape
    n = negative_ids.shape[0]
    np_pad = 1024
    neg_pad = jnp.pad(negative_ids, (0, np_pad - n))
    ids_all = jnp.concatenate([positive_ids, neg_pad])
    table3 = table.reshape(table.shape[0] // 8, 8, d)
    emb_all = _sc_gather_rows(table3, ids_all)
    neg_idf = jnp.pad(negative_ids.astype(jnp.float32).reshape(1, n),
                      ((0, 0), (0, np_pad - n)), constant_values=-1.0)
    return _tc_logits(emb_all, query, neg_idf, n, bb=2048)


# speedup vs baseline: 1.0241x; 1.0130x over previous
import functools

import jax
import jax.numpy as jnp
from jax import lax
from jax.experimental import pallas as pl
from jax.experimental.pallas import tpu as pltpu
from jax.experimental.pallas import tpu_sc as plsc

_FALSE_NEG_SCORE = -1e9
_NW = 32
_W = 128


def _sc_gather_rows(table3, ids):
    n, = ids.shape
    d = table3.shape[2]
    b_per_w = n // _NW
    mesh = plsc.VectorSubcoreMesh(core_axis_name="c", subcore_axis_name="s")

    @functools.partial(
        pl.kernel,
        mesh=mesh,
        out_type=jax.ShapeDtypeStruct((n, _W), jnp.float32),
        scratch_types=[
            pltpu.VMEM((b_per_w,), jnp.int32),
            pltpu.VMEM((b_per_w, _W), jnp.float32),
            pltpu.SemaphoreType.DMA,
        ],
        compiler_params=pltpu.CompilerParams(needs_layout_passes=False),
    )
    def gather_kernel(ids_hbm, table_hbm, out_hbm, ids_v, rows_v, sem):
        wid = lax.axis_index("s") * 2 + lax.axis_index("c")
        base = wid * b_per_w
        pltpu.sync_copy(ids_hbm.at[pl.ds(base, b_per_w)], ids_v)
        lane_iota = lax.iota(jnp.int32, 16)
        id_lane = jnp.full((16,), d, jnp.int32)

        def chunk_body(c, _):
            ids16 = ids_v[pl.ds(c * 16, 16)]
            plsc.store_scatter(rows_v, [c * 16 + lane_iota, id_lane],
                               ids16.astype(jnp.float32))
            for l in range(16):
                idl = jnp.sum(jnp.where(lane_iota == l, ids16, 0))
                t = lax.shift_right_logical(idl, 3)
                s = lax.bitwise_and(idl, 7)
                pltpu.async_copy(table_hbm.at[t, s],
                                 rows_v.at[c * 16 + l, pl.ds(0, d)], sem)
            return ()

        lax.fori_loop(0, b_per_w // 16, chunk_body, (), unroll=False)
        pltpu.make_async_copy(out_hbm.at[pl.ds(base, b_per_w // 2)],
                              rows_v.at[pl.ds(0, b_per_w // 2)], sem).wait()
        pltpu.sync_copy(rows_v, out_hbm.at[pl.ds(base, b_per_w)])

    return gather_kernel(ids, table3)


def _tc_logits(emb_all, query, neg_idf, n_neg, bb):
    b, d = query.shape
    np_pad = neg_idf.shape[1]
    neg_row_block = b // np_pad

    def body(q_ref, pos_ref, neg_ref, nid_ref, out_ref):
        q = q_ref[...]
        p = pos_ref[:, :d]
        pid = pos_ref[:, d:d + 1]
        pos_score = jnp.sum(q * p, axis=1, keepdims=True)
        scores = lax.dot_general(
            q, neg_ref[:, :d], (((1,), (1,)), ((), ())),
            preferred_element_type=jnp.float32)[:, :1 + n_neg]
        mask = pid == nid_ref[:, :1 + n_neg]
        scores = jnp.where(mask, _FALSE_NEG_SCORE, scores)
        col0 = lax.broadcasted_iota(jnp.int32, (bb, 1 + n_neg), 1) == 0
        out_ref[...] = jnp.where(col0, pos_score, scores)

    return pl.pallas_call(
        body,
        grid=(b // bb,),
        in_specs=[
            pl.BlockSpec((bb, d), lambda i: (i, 0)),
            pl.BlockSpec((bb, _W), lambda i: (i, 0)),
            pl.BlockSpec((np_pad, _W), lambda i: (neg_row_block, 0)),
            pl.BlockSpec((1, np_pad), lambda i: (0, 0)),
        ],
        out_specs=pl.BlockSpec((bb, 1 + n_neg), lambda i: (i, 0)),
        out_shape=jax.ShapeDtypeStruct((b, 1 + n_neg), jnp.float32),
    )(query, emb_all, emb_all, neg_idf)


def kernel(query, table, positive_ids, negative_ids):
    b, d = query.shape
    n = negative_ids.shape[0]
    np_pad = 1024
    neg_pad = jnp.pad(negative_ids, (1, np_pad - n - 1))
    ids_all = jnp.concatenate([positive_ids, neg_pad])
    table3 = table.reshape(table.shape[0] // 8, 8, d)
    emb_all = _sc_gather_rows(table3, ids_all)
    neg_idf = jnp.pad(negative_ids.astype(jnp.float32).reshape(1, n),
                      ((0, 0), (1, np_pad - n - 1)), constant_values=-1.0)
    return _tc_logits(emb_all, query, neg_idf, n, bb=2048)
